# Initial kernel scaffold; baseline (speedup 1.0000x reference)
import jax
import jax.numpy as jnp
from jax.experimental import pallas as pl


def kernel(x, edge_index, edge_weight, W1, b1, W2, b2):
    raise NotImplementedError("write your pallas kernel here")


# speedup vs baseline: 6.4278x; 6.4278x over previous
import jax
import jax.numpy as jnp
from jax import lax
from jax.experimental import pallas as pl
from jax.experimental.pallas import tpu as pltpu
from jax.experimental.pallas import tpu_sc as plsc

N = 10000
D = 128
NC = 2
NS = 16
NW = NC * NS
L = 16
CHUNK = 128
STRIPE = 624
N2 = 640


def _mesh():
    return plsc.VectorSubcoreMesh(
        core_axis_name="c", subcore_axis_name="s", num_cores=NC, num_subcores=NS
    )


def _lin1_body(x_ref, w_ref, b_ref, o_ref):
    o_ref[...] = (
        jnp.dot(x_ref[...], w_ref[...], preferred_element_type=jnp.float32)
        + b_ref[...]
    )


def _lin1(x, W1, b1):
    return pl.pallas_call(
        _lin1_body,
        grid=(10,),
        in_specs=[
            pl.BlockSpec((1000, D), lambda i: (i, 0)),
            pl.BlockSpec((D, D), lambda i: (0, 0)),
            pl.BlockSpec((1, D), lambda i: (0, 0)),
        ],
        out_specs=pl.BlockSpec((1000, D), lambda i: (i, 0)),
        out_shape=jax.ShapeDtypeStruct((N, D), jnp.float32),
    )(x, W1, b1.reshape(1, D))


def _spmm_rows_body(h_hbm, src_hbm, dst_hbm, w_hbm, out_hbm,
                    sidx, didx, wv, rows, acc, sem):
    nchunk = src_hbm.shape[0] // (NW * CHUNK)
    cid = lax.axis_index("c")
    sid = lax.axis_index("s")
    wid = sid * NC + cid

    def _zrow(i, _):
        for j in range(D // L):
            rows[i, pl.ds(j * L, L)] = jnp.zeros((L,), jnp.float32)
        return 0

    lax.fori_loop(0, CHUNK, _zrow, 0)
    row0 = pl.multiple_of(sid * STRIPE, 8)

    @pl.when(sid < NS - 1)
    def _():
        for k in range(4):
            pltpu.sync_copy(rows.at[pl.ds(0, CHUNK)],
                            acc.at[pl.ds(row0 + k * CHUNK, CHUNK)])
        pltpu.sync_copy(rows.at[pl.ds(0, 112)],
                        acc.at[pl.ds(row0 + 4 * CHUNK, 112)])

    @pl.when(sid == NS - 1)
    def _():
        for k in range(5):
            pltpu.sync_copy(rows.at[pl.ds(0, CHUNK)],
                            acc.at[pl.ds(row0 + k * CHUNK, CHUNK)])

    plsc.subcore_barrier()

    def _chunk(c, _):
        base = (wid * nchunk + c) * CHUNK
        pltpu.sync_copy(src_hbm.at[pl.ds(base, CHUNK)], sidx)
        pltpu.sync_copy(dst_hbm.at[pl.ds(base, CHUNK)], didx)
        pltpu.sync_copy(w_hbm.at[pl.ds(base, CHUNK)], wv)
        pltpu.async_copy(h_hbm.at[sidx], rows, sem).wait()

        def _scale(g, _):
            wvec = wv[pl.ds(g * L, L)]
            for j in range(L):
                w = wvec[j]
                r = g * L + j
                for k in range(D // L):
                    sl = pl.ds(k * L, L)
                    rows[r, sl] = rows[r, sl] * w
            return 0

        lax.fori_loop(0, CHUNK // L, _scale, 0)
        pltpu.sync_copy(rows, acc.at[didx], add=True)
        return 0

    lax.fori_loop(0, nchunk, _chunk, 0)
    plsc.subcore_barrier()

    def _export(r, n):
        pltpu.sync_copy(acc.at[pl.ds(r, n)], rows.at[pl.ds(0, n)])
        pltpu.sync_copy(rows.at[pl.ds(0, n)], out_hbm.at[cid, pl.ds(r, n)])

    @pl.when(sid < NS - 1)
    def _():
        for k in range(4):
            _export(row0 + k * CHUNK, CHUNK)
        _export(row0 + 4 * CHUNK, 112)

    @pl.when(sid == NS - 1)
    def _():
        for k in range(5):
            _export(row0 + k * CHUNK, CHUNK)


def _spmm_rows(h, src, dst, w):
    run = pl.kernel(
        _spmm_rows_body,
        out_type=jax.ShapeDtypeStruct((NC, N, D), jnp.float32),
        mesh=_mesh(),
        scratch_types=[
            pltpu.VMEM((CHUNK,), jnp.int32),
            pltpu.VMEM((CHUNK,), jnp.int32),
            pltpu.VMEM((CHUNK,), jnp.float32),
            pltpu.VMEM((CHUNK, D), jnp.float32),
            pltpu.VMEM_SHARED((N, D), jnp.float32),
            pltpu.SemaphoreType.DMA,
        ],
    )
    return run(h, src, dst, w)


def _lin2_body(p_ref, w_ref, b_ref, o_ref):
    g = jnp.maximum(p_ref[0] + p_ref[1], 0.0)
    o_ref[...] = (
        jnp.dot(g, w_ref[...], preferred_element_type=jnp.float32) + b_ref[...]
    )


def _lin2(p, W2, b2):
    return pl.pallas_call(
        _lin2_body,
        grid=(10,),
        in_specs=[
            pl.BlockSpec((NC, 1000, D), lambda i: (0, i, 0)),
            pl.BlockSpec((D, 1), lambda i: (0, 0)),
            pl.BlockSpec((1, 1), lambda i: (0, 0)),
        ],
        out_specs=pl.BlockSpec((1000, 1), lambda i: (i, 0)),
        out_shape=jax.ShapeDtypeStruct((N, 1), jnp.float32),
    )(p, W2, b2.reshape(1, 1))


def _spmm_scalar_body(y_hbm, src_hbm, dst_hbm, w_hbm, out_hbm,
                      yv, accv, sidx, didx, wv):
    nchunk = src_hbm.shape[0] // (NW * CHUNK)
    cid = lax.axis_index("c")
    sid = lax.axis_index("s")
    wid = sid * NC + cid

    def _zrow(i, _):
        accv[i] = jnp.zeros((L,), jnp.float32)
        return 0

    lax.fori_loop(0, N2, _zrow, 0)
    pltpu.sync_copy(y_hbm, yv)

    def _chunk(c, _):
        base = (wid * nchunk + c) * CHUNK
        pltpu.sync_copy(src_hbm.at[pl.ds(base, CHUNK)], sidx)
        pltpu.sync_copy(dst_hbm.at[pl.ds(base, CHUNK)], didx)
        pltpu.sync_copy(w_hbm.at[pl.ds(base, CHUNK)], wv)
        for k in range(CHUNK // L):
            sl = pl.ds(k * L, L)
            idx = sidx[sl]
            vals = plsc.load_gather(yv, [idx]) * wv[sl]
            d = didx[sl]
            dr = lax.shift_right_logical(d, 4)
            dc = lax.bitwise_and(d, 15)
            plsc.addupdate_scatter(accv, [dr, dc], vals)
        return 0

    lax.fori_loop(0, nchunk, _chunk, 0)

    pltpu.sync_copy(accv,
                    out_hbm.at[pl.ds(pl.multiple_of(wid * N2, 8), N2)])


def _spmm_scalar(y, src, dst, w):
    run = pl.kernel(
        _spmm_scalar_body,
        out_type=jax.ShapeDtypeStruct((NW * N2, L), jnp.float32),
        mesh=_mesh(),
        compiler_params=pltpu.CompilerParams(needs_layout_passes=False),
        scratch_types=[
            pltpu.VMEM((N,), jnp.float32),
            pltpu.VMEM((N2, L), jnp.float32),
            pltpu.VMEM((CHUNK,), jnp.int32),
            pltpu.VMEM((CHUNK,), jnp.int32),
            pltpu.VMEM((CHUNK,), jnp.float32),
        ],
    )
    return run(y, src, dst, w)


def _psum_body(q_ref, o_ref):
    o_ref[...] = jnp.sum(q_ref[...], axis=0)


def _psum(q):
    return pl.pallas_call(
        _psum_body,
        in_specs=[pl.BlockSpec((NW, N2, L), lambda: (0, 0, 0))],
        out_specs=pl.BlockSpec((N2, L), lambda: (0, 0)),
        out_shape=jax.ShapeDtypeStruct((N2, L), jnp.float32),
    )(q)


def kernel(x, edge_index, edge_weight, W1, b1, W2, b2):
    e = edge_index.shape[1]
    nchunk = -(-e // (NW * CHUNK))
    pad = NW * CHUNK * nchunk - e
    src = jnp.pad(edge_index[0].astype(jnp.int32), (0, pad))
    dst = jnp.pad(edge_index[1].astype(jnp.int32), (0, pad))
    w = jnp.pad(edge_weight.astype(jnp.float32), (0, pad))

    h1 = _lin1(x, W1, b1)
    p = _spmm_rows(h1, src, dst, w)
    y = _lin2(p, W2, b2)
    q = _spmm_scalar(y.reshape(N), src, dst, w)
    out = _psum(q.reshape(NW, N2, L))
    return out.reshape(N2 * L)[:N].reshape(N, 1)


# speedup vs baseline: 8.6546x; 1.3464x over previous
import jax
import jax.numpy as jnp
from jax import lax
from jax.experimental import pallas as pl
from jax.experimental.pallas import tpu as pltpu
from jax.experimental.pallas import tpu_sc as plsc

N = 10000
D = 128
NC = 2
NS = 16
NW = NC * NS
L = 16
CHUNK = 96
STRIPE = 624
N2 = 640


def _stripe_sizes(total):
    out, r = [], total
    while r > 0:
        out.append(min(CHUNK, r))
        r -= out[-1]
    return out


def _mesh():
    return plsc.VectorSubcoreMesh(
        core_axis_name="c", subcore_axis_name="s", num_cores=NC, num_subcores=NS
    )


def _lin1_body(x_ref, w_ref, b_ref, o_ref):
    o_ref[...] = (
        jnp.dot(x_ref[...], w_ref[...], preferred_element_type=jnp.float32)
        + b_ref[...]
    )


def _lin1(x, W1, b1):
    return pl.pallas_call(
        _lin1_body,
        grid=(10,),
        in_specs=[
            pl.BlockSpec((1000, D), lambda i: (i, 0)),
            pl.BlockSpec((D, D), lambda i: (0, 0)),
            pl.BlockSpec((1, D), lambda i: (0, 0)),
        ],
        out_specs=pl.BlockSpec((1000, D), lambda i: (i, 0)),
        out_shape=jax.ShapeDtypeStruct((N, D), jnp.float32),
    )(x, W1, b1.reshape(1, D))


def _spmm_rows_body(h_hbm, src_hbm, dst_hbm, w_hbm, out_hbm,
                    sall, wall, didx0, didx1, rows0, rows1, acc,
                    gsem0, gsem1, ssem0, ssem1, dsem0, dsem1):
    nchunk = src_hbm.shape[0] // (NW * CHUNK)
    cid = lax.axis_index("c")
    sid = lax.axis_index("s")
    wid = sid * NC + cid
    ebase = pl.multiple_of(wid * nchunk * CHUNK, 8)

    pltpu.sync_copy(src_hbm.at[pl.ds(ebase, nchunk * CHUNK)], sall)
    pltpu.sync_copy(w_hbm.at[pl.ds(ebase, nchunk * CHUNK)], wall)

    def _zrow(i, _):
        for j in range(D // L):
            rows0[i, pl.ds(j * L, L)] = jnp.zeros((L,), jnp.float32)
        return 0

    lax.fori_loop(0, CHUNK, _zrow, 0)
    row0 = pl.multiple_of(sid * STRIPE, 8)

    @pl.when(sid < NS - 1)
    def _():
        off = 0
        for n in _stripe_sizes(STRIPE):
            pltpu.sync_copy(rows0.at[pl.ds(0, n)],
                            acc.at[pl.ds(row0 + off, n)])
            off += n

    @pl.when(sid == NS - 1)
    def _():
        off = 0
        for n in _stripe_sizes(N - (NS - 1) * STRIPE):
            pltpu.sync_copy(rows0.at[pl.ds(0, n)],
                            acc.at[pl.ds(row0 + off, n)])
            off += n

    plsc.subcore_barrier()

    bufs = ((rows0, didx0, gsem0, ssem0, dsem0),
            (rows1, didx1, gsem1, ssem1, dsem1))

    def _issue(c, b):
        rows, didx, gsem, _, dsem = bufs[b]
        pltpu.async_copy(h_hbm.at[sall.at[pl.ds(c * CHUNK, CHUNK)]],
                         rows, gsem)
        pltpu.async_copy(dst_hbm.at[pl.ds(ebase + c * CHUNK, CHUNK)],
                         didx, dsem)

    def _scale(buf, c):
        rows = bufs[buf][0]
        wbase = pl.multiple_of(c * CHUNK, 8)

        def _grp(g, _):
            wvec = wall[pl.ds(wbase + g * L, L)]
            for j in range(L):
                w = wvec[j]
                r = g * L + j
                for k in range(D // L):
                    sl = pl.ds(k * L, L)
                    rows[r, sl] = rows[r, sl] * w
            return 0

        lax.fori_loop(0, CHUNK // L, _grp, 0)

    def _wait_g(b):
        rows, didx, gsem, _, dsem = bufs[b]
        pltpu.make_async_copy(h_hbm.at[sall.at[pl.ds(0, CHUNK)]],
                              rows, gsem).wait()
        pltpu.make_async_copy(dst_hbm.at[pl.ds(0, CHUNK)], didx, dsem).wait()

    def _wait_s(b):
        rows, didx, _, ssem, _ = bufs[b]
        pltpu.make_async_copy(rows, acc.at[didx], ssem).wait()

    def _scatter(b):
        rows, didx, _, ssem, _ = bufs[b]
        pltpu.async_copy(rows, acc.at[didx], ssem, add=True)

    _issue(0, 0)

    def _pair(c0, _):
        @pl.when(c0 > 0)
        def _():
            _wait_s(1)
        _issue(c0 + 1, 1)
        _wait_g(0)
        _scale(0, c0)
        _scatter(0)
        @pl.when(c0 < nchunk - 2)
        def _():
            _wait_s(0)
            _issue(c0 + 2, 0)
        _wait_g(1)
        _scale(1, c0 + 1)
        _scatter(1)
        return 0

    lax.fori_loop(0, nchunk // 2, lambda i, car: _pair(i * 2, car), 0)
    _wait_s(0)
    _wait_s(1)
    plsc.subcore_barrier()

    def _export(r, n):
        pltpu.sync_copy(acc.at[pl.ds(r, n)], rows0.at[pl.ds(0, n)])
        pltpu.sync_copy(rows0.at[pl.ds(0, n)], out_hbm.at[cid, pl.ds(r, n)])

    @pl.when(sid < NS - 1)
    def _():
        off = 0
        for n in _stripe_sizes(STRIPE):
            _export(row0 + off, n)
            off += n

    @pl.when(sid == NS - 1)
    def _():
        off = 0
        for n in _stripe_sizes(N - (NS - 1) * STRIPE):
            _export(row0 + off, n)
            off += n


def _spmm_rows(h, src, dst, w):
    nchunk = src.shape[0] // (NW * CHUNK)
    run = pl.kernel(
        _spmm_rows_body,
        out_type=jax.ShapeDtypeStruct((NC, N, D), jnp.float32),
        mesh=_mesh(),
        scratch_types=[
            pltpu.VMEM((nchunk * CHUNK,), jnp.int32),
            pltpu.VMEM((nchunk * CHUNK,), jnp.float32),
            pltpu.VMEM((CHUNK,), jnp.int32),
            pltpu.VMEM((CHUNK,), jnp.int32),
            pltpu.VMEM((CHUNK, D), jnp.float32),
            pltpu.VMEM((CHUNK, D), jnp.float32),
            pltpu.VMEM_SHARED((N, D), jnp.float32),
            pltpu.SemaphoreType.DMA,
            pltpu.SemaphoreType.DMA,
            pltpu.SemaphoreType.DMA,
            pltpu.SemaphoreType.DMA,
            pltpu.SemaphoreType.DMA,
            pltpu.SemaphoreType.DMA,
        ],
    )
    return run(h, src, dst, w)


def _lin2_body(p_ref, w_ref, b_ref, o_ref):
    g = jnp.maximum(p_ref[0] + p_ref[1], 0.0)
    o_ref[...] = (
        jnp.dot(g, w_ref[...], preferred_element_type=jnp.float32) + b_ref[...]
    )


def _lin2(p, W2, b2):
    return pl.pallas_call(
        _lin2_body,
        grid=(10,),
        in_specs=[
            pl.BlockSpec((NC, 1000, D), lambda i: (0, i, 0)),
            pl.BlockSpec((D, 1), lambda i: (0, 0)),
            pl.BlockSpec((1, 1), lambda i: (0, 0)),
        ],
        out_specs=pl.BlockSpec((1000, 1), lambda i: (i, 0)),
        out_shape=jax.ShapeDtypeStruct((N, 1), jnp.float32),
    )(p, W2, b2.reshape(1, 1))


def _spmm_scalar_body(y_hbm, src_hbm, dst_hbm, w_hbm, out_hbm,
                      yv, accv, sall, dall, wall):
    nedge_w = src_hbm.shape[0] // NW
    cid = lax.axis_index("c")
    sid = lax.axis_index("s")
    wid = sid * NC + cid
    ebase = pl.multiple_of(wid * nedge_w, 8)

    pltpu.sync_copy(src_hbm.at[pl.ds(ebase, nedge_w)], sall)
    pltpu.sync_copy(dst_hbm.at[pl.ds(ebase, nedge_w)], dall)
    pltpu.sync_copy(w_hbm.at[pl.ds(ebase, nedge_w)], wall)
    pltpu.sync_copy(y_hbm, yv)

    def _zrow(i, _):
        accv[i] = jnp.zeros((L,), jnp.float32)
        return 0

    lax.fori_loop(0, N2, _zrow, 0)

    def _grp(g, _):
        sl = pl.ds(g * L, L)
        idx = sall[sl]
        vals = plsc.load_gather(yv, [idx]) * wall[sl]
        d = dall[sl]
        dr = lax.shift_right_logical(d, 4)
        dc = lax.bitwise_and(d, 15)
        plsc.addupdate_scatter(accv, [dr, dc], vals)
        return 0

    lax.fori_loop(0, nedge_w // L, _grp, 0)

    pltpu.sync_copy(accv,
                    out_hbm.at[pl.ds(pl.multiple_of(wid * N2, 8), N2)])


def _spmm_scalar(y, src, dst, w):
    run = pl.kernel(
        _spmm_scalar_body,
        out_type=jax.ShapeDtypeStruct((NW * N2, L), jnp.float32),
        mesh=_mesh(),
        compiler_params=pltpu.CompilerParams(needs_layout_passes=False),
        scratch_types=[
            pltpu.VMEM((N,), jnp.float32),
            pltpu.VMEM((N2, L), jnp.float32),
            pltpu.VMEM((src.shape[0] // NW,), jnp.int32),
            pltpu.VMEM((src.shape[0] // NW,), jnp.int32),
            pltpu.VMEM((src.shape[0] // NW,), jnp.float32),
        ],
    )
    return run(y, src, dst, w)


def _psum_body(q_ref, o_ref):
    o_ref[...] = jnp.sum(q_ref[...], axis=0)


def _psum(q):
    return pl.pallas_call(
        _psum_body,
        in_specs=[pl.BlockSpec((NW, N2, L), lambda: (0, 0, 0))],
        out_specs=pl.BlockSpec((N2, L), lambda: (0, 0)),
        out_shape=jax.ShapeDtypeStruct((N2, L), jnp.float32),
    )(q)


def kernel(x, edge_index, edge_weight, W1, b1, W2, b2):
    e = edge_index.shape[1]
    nchunk = -(-e // (NW * CHUNK))
    nchunk += nchunk % 2
    pad = NW * CHUNK * nchunk - e
    src = jnp.pad(edge_index[0].astype(jnp.int32), (0, pad))
    dst = jnp.pad(edge_index[1].astype(jnp.int32), (0, pad))
    w = jnp.pad(edge_weight.astype(jnp.float32), (0, pad))

    h1 = _lin1(x, W1, b1)
    p = _spmm_rows(h1, src, dst, w)
    y = _lin2(p, W2, b2)
    q = _spmm_scalar(y.reshape(N), src, dst, w)
    out = _psum(q.reshape(NW, N2, L))
    return out.reshape(N2 * L)[:N].reshape(N, 1)


# speedup vs baseline: 15.3939x; 1.7787x over previous
import functools

import jax
import jax.numpy as jnp
from jax import lax
from jax.experimental import pallas as pl
from jax.experimental.pallas import tpu as pltpu
from jax.experimental.pallas import tpu_sc as plsc

N = 10000
D = 128
NC = 2
NS = 16
NW = NC * NS
L = 16
CHUNK = 96
STRIPE = 624
N2 = 640
NCKA = 158
NCKB = 52


def _stripe_sizes(total):
    out, r = [], total
    while r > 0:
        out.append(min(CHUNK, r))
        r -= out[-1]
    return out


def _mesh():
    return plsc.VectorSubcoreMesh(
        core_axis_name="c", subcore_axis_name="s", num_cores=NC, num_subcores=NS
    )


def _lin1_body(x_ref, w_ref, b_ref, o_ref):
    o_ref[...] = (
        jnp.dot(x_ref[...], w_ref[...], preferred_element_type=jnp.float32)
        + b_ref[...]
    )


def _lin1(x, W1, b1):
    return pl.pallas_call(
        _lin1_body,
        grid=(10,),
        in_specs=[
            pl.BlockSpec((1000, D), lambda i: (i, 0)),
            pl.BlockSpec((D, D), lambda i: (0, 0)),
            pl.BlockSpec((1, D), lambda i: (0, 0)),
        ],
        out_specs=pl.BlockSpec((1000, D), lambda i: (i, 0)),
        out_shape=jax.ShapeDtypeStruct((N, D), jnp.float32),
    )(x, W1, b1.reshape(1, D))


def _spmm_rows_body(h_hbm, src_hbm, dst_hbm, w_hbm, out_hbm,
                    sall, didx0, didx1, wv0, wv1, rows0, rows1, acc,
                    gsem0, gsem1, ssem0, ssem1, dsem0, dsem1,
                    wsem0, wsem1, ncka=0, nckb=0):
    cid = lax.axis_index("c")
    sid = lax.axis_index("s")

    myn = jnp.where(cid == 0, ncka, nckb)
    cstart = jnp.where(cid == 0, sid * ncka, NS * ncka + sid * nckb)
    ebase = pl.multiple_of(cstart * CHUNK, 8)

    @pl.when(cid == 0)
    def _():
        pltpu.sync_copy(src_hbm.at[pl.ds(ebase, ncka * CHUNK)],
                        sall.at[pl.ds(0, ncka * CHUNK)])

    @pl.when(cid == 1)
    def _():
        pltpu.sync_copy(src_hbm.at[pl.ds(ebase, nckb * CHUNK)],
                        sall.at[pl.ds(0, nckb * CHUNK)])

    def _zrow(i, _):
        for j in range(D // L):
            rows0[i, pl.ds(j * L, L)] = jnp.zeros((L,), jnp.float32)
        return 0

    lax.fori_loop(0, CHUNK, _zrow, 0)
    row0 = pl.multiple_of(sid * STRIPE, 8)

    @pl.when(sid < NS - 1)
    def _():
        off = 0
        for n in _stripe_sizes(STRIPE):
            pltpu.sync_copy(rows0.at[pl.ds(0, n)],
                            acc.at[pl.ds(row0 + off, n)])
            off += n

    @pl.when(sid == NS - 1)
    def _():
        off = 0
        for n in _stripe_sizes(N - (NS - 1) * STRIPE):
            pltpu.sync_copy(rows0.at[pl.ds(0, n)],
                            acc.at[pl.ds(row0 + off, n)])
            off += n

    plsc.subcore_barrier()

    bufs = ((rows0, didx0, wv0, gsem0, ssem0, dsem0, wsem0),
            (rows1, didx1, wv1, gsem1, ssem1, dsem1, wsem1))

    def _issue(c, b):
        rows, didx, wv, gsem, _, dsem, wsem = bufs[b]
        pltpu.async_copy(h_hbm.at[sall.at[pl.ds(c * CHUNK, CHUNK)]],
                         rows, gsem)
        pltpu.async_copy(dst_hbm.at[pl.ds(ebase + c * CHUNK, CHUNK)],
                         didx, dsem)
        pltpu.async_copy(w_hbm.at[pl.ds(ebase + c * CHUNK, CHUNK)],
                         wv, wsem)

    def _scale(b):
        rows, _, wv = bufs[b][:3]

        def _grp(g, _):
            wvec = wv[pl.ds(g * L, L)]
            for j in range(L):
                w = wvec[j]
                r = g * L + j
                for k in range(D // L):
                    sl = pl.ds(k * L, L)
                    rows[r, sl] = rows[r, sl] * w
            return 0

        lax.fori_loop(0, CHUNK // L, _grp, 0)

    def _wait_g(b):
        rows, didx, wv, gsem, _, dsem, wsem = bufs[b]
        pltpu.make_async_copy(h_hbm.at[sall.at[pl.ds(0, CHUNK)]],
                              rows, gsem).wait()
        pltpu.make_async_copy(dst_hbm.at[pl.ds(0, CHUNK)], didx, dsem).wait()
        pltpu.make_async_copy(w_hbm.at[pl.ds(0, CHUNK)], wv, wsem).wait()

    def _wait_s(b):
        rows, didx = bufs[b][:2]
        ssem = bufs[b][4]
        pltpu.make_async_copy(rows, acc.at[didx], ssem).wait()

    def _scatter(b):
        rows, didx = bufs[b][:2]
        ssem = bufs[b][4]
        pltpu.async_copy(rows, acc.at[didx], ssem, add=True)

    _issue(0, 0)

    def _pair(c0, _):
        @pl.when(c0 > 0)
        def _():
            _wait_s(1)
        _issue(c0 + 1, 1)
        _wait_g(0)
        _scale(0)
        _scatter(0)
        @pl.when(c0 < myn - 2)
        def _():
            _wait_s(0)
            _issue(c0 + 2, 0)
        _wait_g(1)
        _scale(1)
        _scatter(1)
        return 0

    lax.fori_loop(0, myn // 2, lambda i, car: _pair(i * 2, car), 0)
    _wait_s(0)
    _wait_s(1)
    plsc.subcore_barrier()

    def _export(r, n):
        pltpu.sync_copy(acc.at[pl.ds(r, n)], rows0.at[pl.ds(0, n)])
        pltpu.sync_copy(rows0.at[pl.ds(0, n)], out_hbm.at[cid, pl.ds(r, n)])

    @pl.when(sid < NS - 1)
    def _():
        off = 0
        for n in _stripe_sizes(STRIPE):
            _export(row0 + off, n)
            off += n

    @pl.when(sid == NS - 1)
    def _():
        off = 0
        for n in _stripe_sizes(N - (NS - 1) * STRIPE):
            _export(row0 + off, n)
            off += n


def _spmm_rows(h, src, dst, w):
    ncmax = max(NCKA, NCKB)
    run = pl.kernel(
        functools.partial(_spmm_rows_body, ncka=NCKA, nckb=NCKB),
        out_type=jax.ShapeDtypeStruct((NC, N, D), jnp.float32),
        mesh=_mesh(),
        scratch_types=[
            pltpu.VMEM((ncmax * CHUNK,), jnp.int32),
            pltpu.VMEM((CHUNK,), jnp.int32),
            pltpu.VMEM((CHUNK,), jnp.int32),
            pltpu.VMEM((CHUNK,), jnp.float32),
            pltpu.VMEM((CHUNK,), jnp.float32),
            pltpu.VMEM((CHUNK, D), jnp.float32),
            pltpu.VMEM((CHUNK, D), jnp.float32),
            pltpu.VMEM_SHARED((N, D), jnp.float32),
            pltpu.SemaphoreType.DMA,
            pltpu.SemaphoreType.DMA,
            pltpu.SemaphoreType.DMA,
            pltpu.SemaphoreType.DMA,
            pltpu.SemaphoreType.DMA,
            pltpu.SemaphoreType.DMA,
            pltpu.SemaphoreType.DMA,
            pltpu.SemaphoreType.DMA,
        ],
    )
    return run(h, src, dst, w)


def _lin2_body(p_ref, w_ref, b_ref, o_ref):
    g = jnp.maximum(p_ref[0] + p_ref[1], 0.0)
    o_ref[...] = (
        jnp.dot(g, w_ref[...], preferred_element_type=jnp.float32) + b_ref[...]
    )


def _lin2(p, W2, b2):
    return pl.pallas_call(
        _lin2_body,
        grid=(10,),
        in_specs=[
            pl.BlockSpec((NC, 1000, D), lambda i: (0, i, 0)),
            pl.BlockSpec((D, 1), lambda i: (0, 0)),
            pl.BlockSpec((1, 1), lambda i: (0, 0)),
        ],
        out_specs=pl.BlockSpec((1000, 1), lambda i: (i, 0)),
        out_shape=jax.ShapeDtypeStruct((N, 1), jnp.float32),
    )(p, W2, b2.reshape(1, 1))


def _spmm_scalar_body(y_hbm, src_hbm, dst_hbm, w_hbm, out_hbm,
                      yv, accv, sall, dall, wall):
    nedge_w = src_hbm.shape[0] // NW
    cid = lax.axis_index("c")
    sid = lax.axis_index("s")
    wid = sid * NC + cid
    ebase = pl.multiple_of(wid * nedge_w, 8)

    pltpu.sync_copy(src_hbm.at[pl.ds(ebase, nedge_w)], sall)
    pltpu.sync_copy(dst_hbm.at[pl.ds(ebase, nedge_w)], dall)
    pltpu.sync_copy(w_hbm.at[pl.ds(ebase, nedge_w)], wall)
    pltpu.sync_copy(y_hbm, yv)

    def _zrow(i, _):
        accv[i] = jnp.zeros((L,), jnp.float32)
        return 0

    lax.fori_loop(0, N2, _zrow, 0)

    def _grp(g, _):
        sl = pl.ds(g * L, L)
        idx = sall[sl]
        vals = plsc.load_gather(yv, [idx]) * wall[sl]
        d = dall[sl]
        dr = lax.shift_right_logical(d, 4)
        dc = lax.bitwise_and(d, 15)
        plsc.addupdate_scatter(accv, [dr, dc], vals)
        return 0

    lax.fori_loop(0, nedge_w // L, _grp, 0)

    pltpu.sync_copy(accv,
                    out_hbm.at[pl.ds(pl.multiple_of(wid * N2, 8), N2)])


def _spmm_scalar(y, src, dst, w):
    run = pl.kernel(
        _spmm_scalar_body,
        out_type=jax.ShapeDtypeStruct((NW * N2, L), jnp.float32),
        mesh=_mesh(),
        compiler_params=pltpu.CompilerParams(needs_layout_passes=False),
        scratch_types=[
            pltpu.VMEM((N,), jnp.float32),
            pltpu.VMEM((N2, L), jnp.float32),
            pltpu.VMEM((src.shape[0] // NW,), jnp.int32),
            pltpu.VMEM((src.shape[0] // NW,), jnp.int32),
            pltpu.VMEM((src.shape[0] // NW,), jnp.float32),
        ],
    )
    return run(y, src, dst, w)


def _psum_body(q_ref, o_ref):
    o_ref[...] = jnp.sum(q_ref[...], axis=0)


def _psum(q):
    return pl.pallas_call(
        _psum_body,
        in_specs=[pl.BlockSpec((NW, N2, L), lambda: (0, 0, 0))],
        out_specs=pl.BlockSpec((N2, L), lambda: (0, 0)),
        out_shape=jax.ShapeDtypeStruct((N2, L), jnp.float32),
    )(q)


def kernel(x, edge_index, edge_weight, W1, b1, W2, b2):
    e = edge_index.shape[1]
    pad = NS * (NCKA + NCKB) * CHUNK - e
    src = jnp.pad(edge_index[0].astype(jnp.int32), (0, pad))
    dst = jnp.pad(edge_index[1].astype(jnp.int32), (0, pad))
    w = jnp.pad(edge_weight.astype(jnp.float32), (0, pad))

    h1 = _lin1(x, W1, b1)
    p = _spmm_rows(h1, src, dst, w)
    y = _lin2(p, W2, b2)
    q = _spmm_scalar(y.reshape(N), src, dst, w)
    out = _psum(q.reshape(NW, N2, L))
    return out.reshape(N2 * L)[:N].reshape(N, 1)


# speedup vs baseline: 15.4613x; 1.0044x over previous
import functools

import jax
import jax.numpy as jnp
from jax import lax
from jax.experimental import pallas as pl
from jax.experimental.pallas import tpu as pltpu
from jax.experimental.pallas import tpu_sc as plsc

N = 10000
D = 128
NC = 2
NS = 16
NW = NC * NS
L = 16
CHUNK = 96
STRIPE = 624
N2 = 640
NCKA = 158
NCKB = 52


def _stripe_sizes(total):
    out, r = [], total
    while r > 0:
        out.append(min(CHUNK, r))
        r -= out[-1]
    return out


def _mesh():
    return plsc.VectorSubcoreMesh(
        core_axis_name="c", subcore_axis_name="s", num_cores=NC, num_subcores=NS
    )


def _lin1_body(x_ref, w_ref, b_ref, o_ref):
    o_ref[...] = (
        jnp.dot(x_ref[...], w_ref[...], preferred_element_type=jnp.float32)
        + b_ref[...]
    )


def _lin1(x, W1, b1):
    return pl.pallas_call(
        _lin1_body,
        grid=(10,),
        in_specs=[
            pl.BlockSpec((1000, D), lambda i: (i, 0)),
            pl.BlockSpec((D, D), lambda i: (0, 0)),
            pl.BlockSpec((1, D), lambda i: (0, 0)),
        ],
        out_specs=pl.BlockSpec((1000, D), lambda i: (i, 0)),
        out_shape=jax.ShapeDtypeStruct((N, D), jnp.float32),
    )(x, W1, b1.reshape(1, D))


def _spmm_rows_body(h_hbm, src_hbm, dst_hbm, w_hbm, out_hbm,
                    sall, didx0, didx1, wv0, wv1, rows0, rows1, acc,
                    gsem0, gsem1, ssem0, ssem1, dsem0, dsem1,
                    wsem0, wsem1, ncka=0, nckb=0):
    cid = lax.axis_index("c")
    sid = lax.axis_index("s")

    myn = jnp.where(cid == 0, ncka, nckb)
    cstart = jnp.where(cid == 0, sid * ncka, NS * ncka + sid * nckb)
    ebase = pl.multiple_of(cstart * CHUNK, 8)

    @pl.when(cid == 0)
    def _():
        pltpu.sync_copy(src_hbm.at[pl.ds(ebase, ncka * CHUNK)],
                        sall.at[pl.ds(0, ncka * CHUNK)])

    @pl.when(cid == 1)
    def _():
        pltpu.sync_copy(src_hbm.at[pl.ds(ebase, nckb * CHUNK)],
                        sall.at[pl.ds(0, nckb * CHUNK)])

    def _zrow(i, _):
        for j in range(D // L):
            rows0[i, pl.ds(j * L, L)] = jnp.zeros((L,), jnp.float32)
        return 0

    lax.fori_loop(0, CHUNK, _zrow, 0)
    row0 = pl.multiple_of(sid * STRIPE, 8)

    @pl.when(sid < NS - 1)
    def _():
        off = 0
        for n in _stripe_sizes(STRIPE):
            pltpu.sync_copy(rows0.at[pl.ds(0, n)],
                            acc.at[pl.ds(row0 + off, n)])
            off += n

    @pl.when(sid == NS - 1)
    def _():
        off = 0
        for n in _stripe_sizes(N - (NS - 1) * STRIPE):
            pltpu.sync_copy(rows0.at[pl.ds(0, n)],
                            acc.at[pl.ds(row0 + off, n)])
            off += n

    plsc.subcore_barrier()

    bufs = ((rows0, didx0, wv0, gsem0, ssem0, dsem0, wsem0),
            (rows1, didx1, wv1, gsem1, ssem1, dsem1, wsem1))

    def _issue(c, b):
        rows, didx, wv, gsem, _, dsem, wsem = bufs[b]
        pltpu.async_copy(h_hbm.at[sall.at[pl.ds(c * CHUNK, CHUNK)]],
                         rows, gsem)
        pltpu.async_copy(dst_hbm.at[pl.ds(ebase + c * CHUNK, CHUNK)],
                         didx, dsem)
        pltpu.async_copy(w_hbm.at[pl.ds(ebase + c * CHUNK, CHUNK)],
                         wv, wsem)

    def _scale(b):
        rows, _, wv = bufs[b][:3]

        @plsc.parallel_loop(0, CHUNK // L, unroll=2)
        def _grp(g):
            wvec = wv[pl.ds(g * L, L)]
            for j in range(L):
                w = wvec[j]
                r = g * L + j
                for k in range(D // L):
                    sl = pl.ds(k * L, L)
                    rows[r, sl] = rows[r, sl] * w

    def _wait_g(b):
        rows, didx, wv, gsem, _, dsem, wsem = bufs[b]
        pltpu.make_async_copy(h_hbm.at[sall.at[pl.ds(0, CHUNK)]],
                              rows, gsem).wait()
        pltpu.make_async_copy(dst_hbm.at[pl.ds(0, CHUNK)], didx, dsem).wait()
        pltpu.make_async_copy(w_hbm.at[pl.ds(0, CHUNK)], wv, wsem).wait()

    def _wait_s(b):
        rows, didx = bufs[b][:2]
        ssem = bufs[b][4]
        pltpu.make_async_copy(rows, acc.at[didx], ssem).wait()

    def _scatter(b):
        rows, didx = bufs[b][:2]
        ssem = bufs[b][4]
        pltpu.async_copy(rows, acc.at[didx], ssem, add=True)

    _issue(0, 0)

    def _pair(c0, _):
        @pl.when(c0 > 0)
        def _():
            _wait_s(1)
        _issue(c0 + 1, 1)
        _wait_g(0)
        _scale(0)
        _scatter(0)
        @pl.when(c0 < myn - 2)
        def _():
            _wait_s(0)
            _issue(c0 + 2, 0)
        _wait_g(1)
        _scale(1)
        _scatter(1)
        return 0

    lax.fori_loop(0, myn // 2, lambda i, car: _pair(i * 2, car), 0)
    _wait_s(0)
    _wait_s(1)
    plsc.subcore_barrier()

    def _export(r, n):
        pltpu.sync_copy(acc.at[pl.ds(r, n)], rows0.at[pl.ds(0, n)])
        pltpu.sync_copy(rows0.at[pl.ds(0, n)], out_hbm.at[cid, pl.ds(r, n)])

    @pl.when(sid < NS - 1)
    def _():
        off = 0
        for n in _stripe_sizes(STRIPE):
            _export(row0 + off, n)
            off += n

    @pl.when(sid == NS - 1)
    def _():
        off = 0
        for n in _stripe_sizes(N - (NS - 1) * STRIPE):
            _export(row0 + off, n)
            off += n


def _spmm_rows(h, src, dst, w):
    ncmax = max(NCKA, NCKB)
    run = pl.kernel(
        functools.partial(_spmm_rows_body, ncka=NCKA, nckb=NCKB),
        out_type=jax.ShapeDtypeStruct((NC, N, D), jnp.float32),
        mesh=_mesh(),
        scratch_types=[
            pltpu.VMEM((ncmax * CHUNK,), jnp.int32),
            pltpu.VMEM((CHUNK,), jnp.int32),
            pltpu.VMEM((CHUNK,), jnp.int32),
            pltpu.VMEM((CHUNK,), jnp.float32),
            pltpu.VMEM((CHUNK,), jnp.float32),
            pltpu.VMEM((CHUNK, D), jnp.float32),
            pltpu.VMEM((CHUNK, D), jnp.float32),
            pltpu.VMEM_SHARED((N, D), jnp.float32),
            pltpu.SemaphoreType.DMA,
            pltpu.SemaphoreType.DMA,
            pltpu.SemaphoreType.DMA,
            pltpu.SemaphoreType.DMA,
            pltpu.SemaphoreType.DMA,
            pltpu.SemaphoreType.DMA,
            pltpu.SemaphoreType.DMA,
            pltpu.SemaphoreType.DMA,
        ],
    )
    return run(h, src, dst, w)


def _lin2_body(p_ref, w_ref, b_ref, o_ref):
    g = jnp.maximum(p_ref[0] + p_ref[1], 0.0)
    o_ref[...] = (
        jnp.dot(g, w_ref[...], preferred_element_type=jnp.float32) + b_ref[...]
    )


def _lin2(p, W2, b2):
    return pl.pallas_call(
        _lin2_body,
        grid=(10,),
        in_specs=[
            pl.BlockSpec((NC, 1000, D), lambda i: (0, i, 0)),
            pl.BlockSpec((D, 1), lambda i: (0, 0)),
            pl.BlockSpec((1, 1), lambda i: (0, 0)),
        ],
        out_specs=pl.BlockSpec((1000, 1), lambda i: (i, 0)),
        out_shape=jax.ShapeDtypeStruct((N, 1), jnp.float32),
    )(p, W2, b2.reshape(1, 1))


def _spmm_scalar_body(y_hbm, src_hbm, dst_hbm, w_hbm, out_hbm,
                      yv, accv, sall, dall, wall):
    nedge_w = src_hbm.shape[0] // NW
    cid = lax.axis_index("c")
    sid = lax.axis_index("s")
    wid = sid * NC + cid
    ebase = pl.multiple_of(wid * nedge_w, 8)

    pltpu.sync_copy(src_hbm.at[pl.ds(ebase, nedge_w)], sall)
    pltpu.sync_copy(dst_hbm.at[pl.ds(ebase, nedge_w)], dall)
    pltpu.sync_copy(w_hbm.at[pl.ds(ebase, nedge_w)], wall)
    pltpu.sync_copy(y_hbm, yv)

    def _zrow(i, _):
        accv[i] = jnp.zeros((L,), jnp.float32)
        return 0

    lax.fori_loop(0, N2, _zrow, 0)

    def _grp(g, _):
        sl = pl.ds(g * L, L)
        idx = sall[sl]
        vals = plsc.load_gather(yv, [idx]) * wall[sl]
        d = dall[sl]
        dr = lax.shift_right_logical(d, 4)
        dc = lax.bitwise_and(d, 15)
        plsc.addupdate_scatter(accv, [dr, dc], vals)
        return 0

    lax.fori_loop(0, nedge_w // L, _grp, 0)

    pltpu.sync_copy(accv,
                    out_hbm.at[pl.ds(pl.multiple_of(wid * N2, 8), N2)])


def _spmm_scalar(y, src, dst, w):
    run = pl.kernel(
        _spmm_scalar_body,
        out_type=jax.ShapeDtypeStruct((NW * N2, L), jnp.float32),
        mesh=_mesh(),
        compiler_params=pltpu.CompilerParams(needs_layout_passes=False),
        scratch_types=[
            pltpu.VMEM((N,), jnp.float32),
            pltpu.VMEM((N2, L), jnp.float32),
            pltpu.VMEM((src.shape[0] // NW,), jnp.int32),
            pltpu.VMEM((src.shape[0] // NW,), jnp.int32),
            pltpu.VMEM((src.shape[0] // NW,), jnp.float32),
        ],
    )
    return run(y, src, dst, w)


def _psum_body(q_ref, o_ref):
    o_ref[...] = jnp.sum(q_ref[...], axis=0)


def _psum(q):
    return pl.pallas_call(
        _psum_body,
        in_specs=[pl.BlockSpec((NW, N2, L), lambda: (0, 0, 0))],
        out_specs=pl.BlockSpec((N2, L), lambda: (0, 0)),
        out_shape=jax.ShapeDtypeStruct((N2, L), jnp.float32),
    )(q)


def kernel(x, edge_index, edge_weight, W1, b1, W2, b2):
    e = edge_index.shape[1]
    pad = NS * (NCKA + NCKB) * CHUNK - e
    src = jnp.pad(edge_index[0].astype(jnp.int32), (0, pad))
    dst = jnp.pad(edge_index[1].astype(jnp.int32), (0, pad))
    w = jnp.pad(edge_weight.astype(jnp.float32), (0, pad))

    h1 = _lin1(x, W1, b1)
    p = _spmm_rows(h1, src, dst, w)
    y = _lin2(p, W2, b2)
    q = _spmm_scalar(y.reshape(N), src, dst, w)
    out = _psum(q.reshape(NW, N2, L))
    return out.reshape(N2 * L)[:N].reshape(N, 1)


# speedup vs baseline: 15.7413x; 1.0181x over previous
import functools

import jax
import jax.numpy as jnp
from jax import lax
from jax.experimental import pallas as pl
from jax.experimental.pallas import tpu as pltpu
from jax.experimental.pallas import tpu_sc as plsc

N = 10000
D = 128
NC = 2
NS = 16
NW = NC * NS
L = 16
CHUNK = 96
STRIPE = 624
N2 = 640
NCKA = 158
NCKB = 52


def _stripe_sizes(total):
    out, r = [], total
    while r > 0:
        out.append(min(CHUNK, r))
        r -= out[-1]
    return out


def _mesh():
    return plsc.VectorSubcoreMesh(
        core_axis_name="c", subcore_axis_name="s", num_cores=NC, num_subcores=NS
    )


def _lin1_body(x_ref, w_ref, b_ref, o_ref):
    o_ref[...] = (
        jnp.dot(x_ref[...], w_ref[...], preferred_element_type=jnp.float32)
        + b_ref[...]
    )


def _lin1(x, W1, b1):
    return pl.pallas_call(
        _lin1_body,
        grid=(10,),
        in_specs=[
            pl.BlockSpec((1000, D), lambda i: (i, 0)),
            pl.BlockSpec((D, D), lambda i: (0, 0)),
            pl.BlockSpec((1, D), lambda i: (0, 0)),
        ],
        out_specs=pl.BlockSpec((1000, D), lambda i: (i, 0)),
        out_shape=jax.ShapeDtypeStruct((N, D), jnp.float32),
    )(x, W1, b1.reshape(1, D))


def _spmm_rows_body(h_hbm, src_hbm, dst_hbm, w_hbm, out_hbm,
                    sall, didx0, didx1, wv0, wv1, rows0, rows1, acc,
                    gsem0, gsem1, ssem0, ssem1, dsem0, dsem1,
                    wsem0, wsem1, psem, ncka=0, nckb=0):
    cid = lax.axis_index("c")
    sid = lax.axis_index("s")

    myn = jnp.where(cid == 0, ncka, nckb)
    cstart = jnp.where(cid == 0, sid * ncka, NS * ncka + sid * nckb)
    ebase = pl.multiple_of(cstart * CHUNK, 8)

    @pl.when(cid == 0)
    def _():
        pltpu.async_copy(src_hbm.at[pl.ds(ebase, ncka * CHUNK)],
                         sall.at[pl.ds(0, ncka * CHUNK)], psem)

    @pl.when(cid == 1)
    def _():
        pltpu.async_copy(src_hbm.at[pl.ds(ebase, nckb * CHUNK)],
                         sall.at[pl.ds(0, nckb * CHUNK)], psem)

    def _zrow(i, _):
        for j in range(D // L):
            rows0[i, pl.ds(j * L, L)] = jnp.zeros((L,), jnp.float32)
        return 0

    lax.fori_loop(0, CHUNK, _zrow, 0)
    row0 = pl.multiple_of(sid * STRIPE, 8)

    @pl.when(sid < NS - 1)
    def _():
        off = 0
        for n in _stripe_sizes(STRIPE):
            pltpu.sync_copy(rows0.at[pl.ds(0, n)],
                            acc.at[pl.ds(row0 + off, n)])
            off += n

    @pl.when(sid == NS - 1)
    def _():
        off = 0
        for n in _stripe_sizes(N - (NS - 1) * STRIPE):
            pltpu.sync_copy(rows0.at[pl.ds(0, n)],
                            acc.at[pl.ds(row0 + off, n)])
            off += n

    @pl.when(cid == 0)
    def _():
        pltpu.make_async_copy(src_hbm.at[pl.ds(ebase, ncka * CHUNK)],
                              sall.at[pl.ds(0, ncka * CHUNK)], psem).wait()

    @pl.when(cid == 1)
    def _():
        pltpu.make_async_copy(src_hbm.at[pl.ds(ebase, nckb * CHUNK)],
                              sall.at[pl.ds(0, nckb * CHUNK)], psem).wait()

    plsc.subcore_barrier()

    bufs = ((rows0, didx0, wv0, gsem0, ssem0, dsem0, wsem0),
            (rows1, didx1, wv1, gsem1, ssem1, dsem1, wsem1))

    def _issue(c, b):
        rows, didx, wv, gsem, _, dsem, wsem = bufs[b]
        pltpu.async_copy(h_hbm.at[sall.at[pl.ds(c * CHUNK, CHUNK)]],
                         rows, gsem)
        pltpu.async_copy(dst_hbm.at[pl.ds(ebase + c * CHUNK, CHUNK)],
                         didx, dsem)
        pltpu.async_copy(w_hbm.at[pl.ds(ebase + c * CHUNK, CHUNK)],
                         wv, wsem)

    def _scale(b):
        rows, _, wv = bufs[b][:3]

        @plsc.parallel_loop(0, CHUNK // L, unroll=2)
        def _grp(g):
            wvec = wv[pl.ds(g * L, L)]
            for j in range(L):
                w = wvec[j]
                r = g * L + j
                for k in range(D // L):
                    sl = pl.ds(k * L, L)
                    rows[r, sl] = rows[r, sl] * w

    def _wait_g(b):
        rows, didx, wv, gsem, _, dsem, wsem = bufs[b]
        pltpu.make_async_copy(h_hbm.at[sall.at[pl.ds(0, CHUNK)]],
                              rows, gsem).wait()
        pltpu.make_async_copy(dst_hbm.at[pl.ds(0, CHUNK)], didx, dsem).wait()
        pltpu.make_async_copy(w_hbm.at[pl.ds(0, CHUNK)], wv, wsem).wait()

    def _wait_s(b):
        rows, didx = bufs[b][:2]
        ssem = bufs[b][4]
        pltpu.make_async_copy(rows, acc.at[didx], ssem).wait()

    def _scatter(b):
        rows, didx = bufs[b][:2]
        ssem = bufs[b][4]
        pltpu.async_copy(rows, acc.at[didx], ssem, add=True)

    _issue(0, 0)

    def _pair(c0, _):
        @pl.when(c0 > 0)
        def _():
            _wait_s(1)
        _issue(c0 + 1, 1)
        _wait_g(0)
        _scale(0)
        _scatter(0)
        @pl.when(c0 < myn - 2)
        def _():
            _wait_s(0)
            _issue(c0 + 2, 0)
        _wait_g(1)
        _scale(1)
        _scatter(1)
        return 0

    lax.fori_loop(0, myn // 2, lambda i, car: _pair(i * 2, car), 0)
    _wait_s(0)
    _wait_s(1)
    plsc.subcore_barrier()

    def _export(r, n):
        pltpu.sync_copy(acc.at[pl.ds(r, n)], rows0.at[pl.ds(0, n)])
        pltpu.sync_copy(rows0.at[pl.ds(0, n)], out_hbm.at[cid, pl.ds(r, n)])

    @pl.when(sid < NS - 1)
    def _():
        off = 0
        for n in _stripe_sizes(STRIPE):
            _export(row0 + off, n)
            off += n

    @pl.when(sid == NS - 1)
    def _():
        off = 0
        for n in _stripe_sizes(N - (NS - 1) * STRIPE):
            _export(row0 + off, n)
            off += n


def _spmm_rows(h, src, dst, w):
    ncmax = max(NCKA, NCKB)
    run = pl.kernel(
        functools.partial(_spmm_rows_body, ncka=NCKA, nckb=NCKB),
        out_type=jax.ShapeDtypeStruct((NC, N, D), jnp.float32),
        mesh=_mesh(),
        scratch_types=[
            pltpu.VMEM((ncmax * CHUNK,), jnp.int32),
            pltpu.VMEM((CHUNK,), jnp.int32),
            pltpu.VMEM((CHUNK,), jnp.int32),
            pltpu.VMEM((CHUNK,), jnp.float32),
            pltpu.VMEM((CHUNK,), jnp.float32),
            pltpu.VMEM((CHUNK, D), jnp.float32),
            pltpu.VMEM((CHUNK, D), jnp.float32),
            pltpu.VMEM_SHARED((N, D), jnp.float32),
            pltpu.SemaphoreType.DMA,
            pltpu.SemaphoreType.DMA,
            pltpu.SemaphoreType.DMA,
            pltpu.SemaphoreType.DMA,
            pltpu.SemaphoreType.DMA,
            pltpu.SemaphoreType.DMA,
            pltpu.SemaphoreType.DMA,
            pltpu.SemaphoreType.DMA,
            pltpu.SemaphoreType.DMA,
        ],
    )
    return run(h, src, dst, w)


def _lin2_body(p_ref, w_ref, b_ref, o_ref):
    g = jnp.maximum(p_ref[0] + p_ref[1], 0.0)
    o_ref[...] = (
        jnp.dot(g, w_ref[...], preferred_element_type=jnp.float32) + b_ref[...]
    )


def _lin2(p, W2, b2):
    return pl.pallas_call(
        _lin2_body,
        grid=(10,),
        in_specs=[
            pl.BlockSpec((NC, 1000, D), lambda i: (0, i, 0)),
            pl.BlockSpec((D, 1), lambda i: (0, 0)),
            pl.BlockSpec((1, 1), lambda i: (0, 0)),
        ],
        out_specs=pl.BlockSpec((1000, 1), lambda i: (i, 0)),
        out_shape=jax.ShapeDtypeStruct((N, 1), jnp.float32),
    )(p, W2, b2.reshape(1, 1))


def _spmm_scalar_body(y_hbm, src_hbm, dst_hbm, w_hbm, out_hbm,
                      yv, accv, sall, dall, wall, stsem):
    nedge_w = src_hbm.shape[0] // NW
    cid = lax.axis_index("c")
    sid = lax.axis_index("s")
    wid = sid * NC + cid
    ebase = pl.multiple_of(wid * nedge_w, 8)

    cs = pltpu.async_copy(src_hbm.at[pl.ds(ebase, nedge_w)], sall, stsem)
    cd = pltpu.async_copy(dst_hbm.at[pl.ds(ebase, nedge_w)], dall, stsem)
    cw = pltpu.async_copy(w_hbm.at[pl.ds(ebase, nedge_w)], wall, stsem)
    cy = pltpu.async_copy(y_hbm, yv, stsem)

    def _zrow(i, _):
        accv[i] = jnp.zeros((L,), jnp.float32)
        return 0

    lax.fori_loop(0, N2, _zrow, 0)
    cs.wait()
    cd.wait()
    cw.wait()
    cy.wait()

    def _grp(g, _):
        sl = pl.ds(g * L, L)
        idx = sall[sl]
        vals = plsc.load_gather(yv, [idx]) * wall[sl]
        d = dall[sl]
        dr = lax.shift_right_logical(d, 4)
        dc = lax.bitwise_and(d, 15)
        plsc.addupdate_scatter(accv, [dr, dc], vals)
        return 0

    lax.fori_loop(0, nedge_w // L, _grp, 0)

    pltpu.sync_copy(accv,
                    out_hbm.at[pl.ds(pl.multiple_of(wid * N2, 8), N2)])


def _spmm_scalar(y, src, dst, w):
    run = pl.kernel(
        _spmm_scalar_body,
        out_type=jax.ShapeDtypeStruct((NW * N2, L), jnp.float32),
        mesh=_mesh(),
        compiler_params=pltpu.CompilerParams(needs_layout_passes=False),
        scratch_types=[
            pltpu.VMEM((N,), jnp.float32),
            pltpu.VMEM((N2, L), jnp.float32),
            pltpu.VMEM((src.shape[0] // NW,), jnp.int32),
            pltpu.VMEM((src.shape[0] // NW,), jnp.int32),
            pltpu.VMEM((src.shape[0] // NW,), jnp.float32),
            pltpu.SemaphoreType.DMA,
        ],
    )
    return run(y, src, dst, w)


def _psum_body(q_ref, o_ref):
    o_ref[...] = jnp.sum(q_ref[...], axis=0)


def _psum(q):
    return pl.pallas_call(
        _psum_body,
        in_specs=[pl.BlockSpec((NW, N2, L), lambda: (0, 0, 0))],
        out_specs=pl.BlockSpec((N2, L), lambda: (0, 0)),
        out_shape=jax.ShapeDtypeStruct((N2, L), jnp.float32),
    )(q)


def kernel(x, edge_index, edge_weight, W1, b1, W2, b2):
    e = edge_index.shape[1]
    pad = NS * (NCKA + NCKB) * CHUNK - e
    src = jnp.pad(edge_index[0].astype(jnp.int32), (0, pad))
    dst = jnp.pad(edge_index[1].astype(jnp.int32), (0, pad))
    w = jnp.pad(edge_weight.astype(jnp.float32), (0, pad))

    h1 = _lin1(x, W1, b1)
    p = _spmm_rows(h1, src, dst, w)
    y = _lin2(p, W2, b2)
    q = _spmm_scalar(y.reshape(N), src, dst, w)
    out = _psum(q.reshape(NW, N2, L))
    return out.reshape(N2 * L)[:N].reshape(N, 1)
